# baseline (speedup 1.0000x reference)
import math

import jax
import jax.numpy as jnp
from jax import lax
from jax.experimental import pallas as pl
from jax.experimental.pallas import tpu as pltpu
from jax.experimental.pallas import tpu_sc as plsc

BATCH = 4096
HIST = 200
DIM = 64
NUM_INDICES = BATCH * HIST
SCALE = math.sqrt(DIM)

NC, NS, LANES = 2, 16, 16
NW = NC * NS
PER_WORKER = NUM_INDICES // NW
IDX_PER_STREAM = 128
CHUNK = 128
NSTREAM = CHUNK // IDX_PER_STREAM
NCHUNK = PER_WORKER // CHUNK
IDX_ROWS = PER_WORKER // IDX_PER_STREAM
TPAIR = 500000
OGRP = CHUNK // 8


def _sc_body(x_hbm, table_hbm, out_hbm, idx_all, q_idx, rows0, rows1,
             ob0, ob1, sg0, sg1, so0, so1):
    wid = lax.axis_index("s") * NC + lax.axis_index("c")
    out_grp_base = wid * (PER_WORKER // 8)
    rows = (rows0, rows1)
    obuf = (ob0, ob1)
    sg = (sg0, sg1)
    so = (so0, so1)

    pltpu.sync_copy(x_hbm.at[pl.ds(wid * IDX_ROWS, IDX_ROWS)], idx_all)

    def fire_gather(c, p):
        for v in range(IDX_PER_STREAM // LANES):
            sl = idx_all[c, pl.ds(v * LANES, LANES)]
            q_idx[p, pl.ds(v * LANES, LANES)] = lax.shift_right_logical(sl, 1)
        pltpu.async_copy(table_hbm.at[q_idx.at[p]], rows[p], sg[p])

    def drain_gather(sem, ref):
        pltpu.make_async_copy(table_hbm.at[pl.ds(0, CHUNK)], ref, sem).wait()

    def drain_out(sem):
        pltpu.make_async_copy(
            out_hbm.at[pl.ds(0, OGRP)], ob0, sem).wait()

    def select_scale(c, p):
        def g_body(g, carry):
            pv = lax.bitwise_and(idx_all[c, pl.ds(g * LANES, LANES)], 1)
            for l in range(LANES):
                r = g * LANES + l
                sel = pv[l] > 0
                for j in range(DIM // LANES):
                    lo = rows[p][r, pl.ds(j * LANES, LANES)]
                    hi = rows[p][r, pl.ds(DIM + j * LANES, LANES)]
                    obuf[p][g * 2 + l // 8, l % 8, pl.ds(j * LANES, LANES)] = (
                        jnp.where(sel, hi, lo) * SCALE)
            return carry

        lax.fori_loop(0, CHUNK // LANES, g_body, 0)

    fire_gather(0, 0)

    def pair_body(k, carry):
        for half in range(2):
            c = 2 * k + half
            p = half
            q = 1 - half
            drain_gather(sg[p], rows[p])
            if half == 0:
                @pl.when(k > 0)
                def _():
                    drain_out(so[q])
                fire_gather(c + 1, q)
            else:
                drain_out(so[q])

                @pl.when(k < NCHUNK // 2 - 1)
                def _():
                    fire_gather(c + 1, q)
            select_scale(c, p)
            pltpu.async_copy(
                obuf[p],
                out_hbm.at[pl.ds(out_grp_base + c * OGRP, OGRP)],
                so[p])
        return carry

    lax.fori_loop(0, NCHUNK // 2, pair_body, 0)
    drain_out(so[1])


@jax.jit
def _embed(x2d, t128):
    mesh = plsc.VectorSubcoreMesh(core_axis_name="c", subcore_axis_name="s")
    run = pl.kernel(
        _sc_body,
        out_type=jax.ShapeDtypeStruct((NUM_INDICES // 8, 8, DIM),
                                      jnp.float32),
        mesh=mesh,
        scratch_types=[
            pltpu.VMEM((IDX_ROWS, IDX_PER_STREAM), jnp.int32),
            pltpu.VMEM((8, IDX_PER_STREAM), jnp.int32),
            pltpu.VMEM((CHUNK, 2 * DIM), jnp.float32),
            pltpu.VMEM((CHUNK, 2 * DIM), jnp.float32),
            pltpu.VMEM((OGRP, 8, DIM), jnp.float32),
            pltpu.VMEM((OGRP, 8, DIM), jnp.float32),
            pltpu.SemaphoreType.DMA,
            pltpu.SemaphoreType.DMA,
            pltpu.SemaphoreType.DMA,
            pltpu.SemaphoreType.DMA,
        ],
        compiler_params=pltpu.CompilerParams(use_tc_tiling_on_sc=True),
    )
    return run(x2d, t128)


def kernel(x, table):
    x2d = x.astype(jnp.int32).reshape(NUM_INDICES // IDX_PER_STREAM,
                                      IDX_PER_STREAM)
    t128 = table.reshape(TPAIR, 2 * DIM)
    out = _embed(x2d, t128)
    return out.reshape(BATCH, HIST, DIM)


# speedup vs baseline: 1.1427x; 1.1427x over previous
import math

import jax
import jax.numpy as jnp
from jax import lax
from jax.experimental import pallas as pl
from jax.experimental.pallas import tpu as pltpu
from jax.experimental.pallas import tpu_sc as plsc

BATCH = 4096
HIST = 200
DIM = 64
NUM_INDICES = BATCH * HIST
SCALE = math.sqrt(DIM)

NC, NS, LANES = 2, 16, 16
NW = NC * NS
PER_WORKER = NUM_INDICES // NW
IDX_PER_STREAM = 128
CHUNK = 512
NSTREAM = CHUNK // IDX_PER_STREAM
NCHUNK = PER_WORKER // CHUNK
IDX_ROWS = PER_WORKER // IDX_PER_STREAM


def _sc_body(x_hbm, table_hbm, out_hbm, idx_all, rows0, rows1,
             sg0, sg1, so0, so1):
    wid = lax.axis_index("s") * NC + lax.axis_index("c")
    out_base = wid * PER_WORKER
    rows = (rows0, rows1)
    sg = (sg0, sg1)
    so = (so0, so1)

    pltpu.sync_copy(x_hbm.at[pl.ds(wid * IDX_ROWS, IDX_ROWS)], idx_all)

    def fire_gather(c, p):
        for j in range(NSTREAM):
            pltpu.async_copy(
                table_hbm.at[idx_all.at[c * NSTREAM + j]],
                rows[p].at[pl.ds(j * IDX_PER_STREAM, IDX_PER_STREAM)],
                sg[p])

    def drain(sem, nbytes_ref):
        pltpu.make_async_copy(
            out_hbm.at[pl.ds(0, CHUNK)], nbytes_ref, sem).wait()

    def scale(p):
        @plsc.parallel_loop(0, CHUNK, step=1, unroll=8)
        def _(i):
            for j in range(DIM // LANES):
                sl = rows[p][i, pl.ds(j * LANES, LANES)]
                rows[p][i, pl.ds(j * LANES, LANES)] = sl * SCALE

    fire_gather(0, 0)

    def pair_body(k, carry):
        for half in range(2):
            c = 2 * k + half
            p = half
            q = 1 - half
            drain(sg[p], rows[p])
            if half == 0:
                @pl.when(k > 0)
                def _():
                    drain(so[q], rows[q])
                fire_gather(c + 1, q)
            else:
                drain(so[q], rows[q])

                @pl.when(k < NCHUNK // 2 - 1)
                def _():
                    fire_gather(c + 1, q)
            scale(p)
            pltpu.async_copy(
                rows[p], out_hbm.at[pl.ds(out_base + c * CHUNK, CHUNK)],
                so[p])
        return carry

    lax.fori_loop(0, NCHUNK // 2, pair_body, 0)
    drain(so[1], rows[1])


@jax.jit
def _embed(x2d, table):
    mesh = plsc.VectorSubcoreMesh(core_axis_name="c", subcore_axis_name="s")
    run = pl.kernel(
        _sc_body,
        out_type=jax.ShapeDtypeStruct((NUM_INDICES, DIM), jnp.float32),
        mesh=mesh,
        scratch_types=[
            pltpu.VMEM((IDX_ROWS, IDX_PER_STREAM), jnp.int32),
            pltpu.VMEM((CHUNK, DIM), jnp.float32),
            pltpu.VMEM((CHUNK, DIM), jnp.float32),
            pltpu.SemaphoreType.DMA,
            pltpu.SemaphoreType.DMA,
            pltpu.SemaphoreType.DMA,
            pltpu.SemaphoreType.DMA,
        ],
        compiler_params=pltpu.CompilerParams(use_tc_tiling_on_sc=False),
    )
    return run(x2d, table)


def kernel(x, table):
    x2d = x.astype(jnp.int32).reshape(NUM_INDICES // IDX_PER_STREAM,
                                      IDX_PER_STREAM)
    out = _embed(x2d, table)
    return out.reshape(BATCH, HIST, DIM)
